# Initial kernel scaffold; baseline (speedup 1.0000x reference)
import jax
import jax.numpy as jnp
from jax.experimental import pallas as pl


def kernel(queries, keys, keys_length, W1, b1, W2, b2, Wd, bd, W_ih, W_hh, b_ih, b_hh):
    raise NotImplementedError("write your pallas kernel here")


# speedup vs baseline: 4.2704x; 4.2704x over previous
import functools

import jax
import jax.numpy as jnp
from jax.experimental import pallas as pl


def _body(T, HS, CH, q_ref, kt_ref, len_ref, w1q_ref, w1k_ref, w1qk_ref,
          b1_ref, w2t_ref, b2_ref, wd_ref, wih_ref, bih_ref, whh_ref,
          bhh_ref, out_ref):
    f32 = jnp.float32
    q = q_ref[...]
    Bb, H = q.shape
    w1k = w1k_ref[...]
    w1qk = w1qk_ref[...]
    w2t = w2t_ref[...]
    b2 = b2_ref[...]
    wd = wd_ref[...]
    qpart = jnp.dot(q, w1q_ref[...], preferred_element_type=f32) + b1_ref[...]
    NA = qpart.shape[1]

    cols = []
    for t0 in range(0, T, CH):
        c = min(CH, T - t0)
        x = kt_ref[t0:t0 + c].reshape(c * Bb, H)
        qt = jnp.broadcast_to(q[None], (c, Bb, H)).reshape(c * Bb, H)
        qpt = jnp.broadcast_to(qpart[None], (c, Bb, NA)).reshape(c * Bb, NA)
        a1 = jax.nn.sigmoid(
            qpt
            + jnp.dot(x, w1k, preferred_element_type=f32)
            + jnp.dot(qt * x, w1qk, preferred_element_type=f32))
        a2 = jax.nn.sigmoid(jnp.dot(a1, w2t, preferred_element_type=f32) + b2)
        s = jnp.sum(a2 * wd, axis=1, keepdims=True)
        for t in range(c):
            cols.append(s[t * Bb:(t + 1) * Bb])
    scores = jnp.concatenate(cols, axis=1)

    valid = jax.lax.broadcasted_iota(jnp.int32, (Bb, T), 1) < len_ref[...]
    scores = jnp.where(valid, scores, f32(-8.8e7))
    scores = scores - jnp.max(scores, axis=1, keepdims=True)
    e = jnp.exp(scores)
    att = e / jnp.sum(e, axis=1, keepdims=True)

    wih = wih_ref[...]
    bih = bih_ref[...]
    whh = whh_ref[...]
    bhh = bhh_ref[...]
    h = jnp.zeros((Bb, HS), f32)
    for t in range(T):
        x = kt_ref[t]
        gi = jnp.dot(x, wih, preferred_element_type=f32) + bih
        gh = jnp.dot(h, whh, preferred_element_type=f32) + bhh
        r = jax.nn.sigmoid(gi[:, :HS] + gh[:, :HS])
        n = jnp.tanh(gi[:, HS:] + r * gh[:, HS:])
        a = att[:, t:t + 1]
        h = (1.0 - a) * h + a * n
    out_ref[...] = h


def kernel(queries, keys, keys_length, W1, b1, W2, b2, Wd, bd, W_ih, W_hh,
           b_ih, b_hh):
    B, T, H = keys.shape
    HS = W_hh.shape[1]
    Bb = 512
    while B % Bb:
        Bb //= 2

    kt = jnp.transpose(keys, (1, 0, 2))
    len2 = jnp.maximum(keys_length.astype(jnp.int32), 1).reshape(B, 1)

    w1a, w1b, w1c, w1d = jnp.split(W1, 4, axis=1)
    w1q = (w1a + w1c).T
    w1k = (w1b - w1c).T
    w1qk = w1d.T
    b1r = b1.reshape(1, -1)
    w2t = W2.T
    b2r = b2.reshape(1, -1)
    wd_s = (Wd / jnp.sqrt(jnp.float32(H))).reshape(1, -1)
    wih2 = jnp.concatenate([W_ih[:HS], W_ih[2 * HS:]], axis=0).T
    whh2 = jnp.concatenate([W_hh[:HS], W_hh[2 * HS:]], axis=0).T
    bih2 = jnp.concatenate([b_ih[:HS], b_ih[2 * HS:]]).reshape(1, -1)
    bhh2 = jnp.concatenate([b_hh[:HS], b_hh[2 * HS:]]).reshape(1, -1)

    full = lambda i: (0, 0)
    out = pl.pallas_call(
        functools.partial(_body, T, HS, 10),
        grid=(B // Bb,),
        in_specs=[
            pl.BlockSpec((Bb, H), lambda i: (i, 0)),
            pl.BlockSpec((T, Bb, H), lambda i: (0, i, 0)),
            pl.BlockSpec((Bb, 1), lambda i: (i, 0)),
            pl.BlockSpec(w1q.shape, full),
            pl.BlockSpec(w1k.shape, full),
            pl.BlockSpec(w1qk.shape, full),
            pl.BlockSpec(b1r.shape, full),
            pl.BlockSpec(w2t.shape, full),
            pl.BlockSpec(b2r.shape, full),
            pl.BlockSpec(wd_s.shape, full),
            pl.BlockSpec(wih2.shape, full),
            pl.BlockSpec(bih2.shape, full),
            pl.BlockSpec(whh2.shape, full),
            pl.BlockSpec(bhh2.shape, full),
        ],
        out_specs=pl.BlockSpec((Bb, HS), lambda i: (i, 0)),
        out_shape=jax.ShapeDtypeStruct((B, HS), jnp.float32),
    )(queries, kt, len2, w1q, w1k, w1qk, b1r, w2t, b2r, wd_s, wih2, bih2,
      whh2, bhh2)
    return out
